# Initial kernel scaffold; baseline (speedup 1.0000x reference)
import jax
import jax.numpy as jnp
from jax.experimental import pallas as pl


def kernel(x, emb, W1, b1, W2, b2, W3, b3):
    raise NotImplementedError("write your pallas kernel here")


# speedup vs baseline: 1.2549x; 1.2549x over previous
import functools

import jax
import jax.numpy as jnp
from jax import lax
from jax.experimental import pallas as pl
from jax.experimental.pallas import tpu as pltpu
from jax.experimental.pallas import tpu_sc as plsc

B = 16384
V = 64
D = 4

_SC_INFO = plsc.get_sparse_core_info()
_NC = _SC_INFO.num_cores
_NS = _SC_INFO.num_subcores
_NW = _NC * _NS
_L = _SC_INFO.num_lanes
_BPW = B // _NW
_GRP = _BPW // _L


def _gelu_exact(h):
    return 0.5 * h * (1.0 + lax.erf(h * 0.7071067811865476))


def _mlp_table_kernel(emb_ref, w1_ref, b1_ref, w2_ref, b2_ref, w3_ref,
                      b3_ref, out_ref):
    h = emb_ref[...]
    h = jnp.dot(h, w1_ref[...], preferred_element_type=jnp.float32) + b1_ref[...]
    h = _gelu_exact(h)
    h = jnp.dot(h, w2_ref[...], preferred_element_type=jnp.float32) + b2_ref[...]
    h = _gelu_exact(h)
    h = jnp.dot(h, w3_ref[...], preferred_element_type=jnp.float32) + b3_ref[...]
    out_ref[...] = h


def _compute_table(emb, W1, b1, W2, b2, W3, b3):
    return pl.pallas_call(
        _mlp_table_kernel,
        out_shape=jax.ShapeDtypeStruct((V, D), jnp.float32),
    )(emb, W1, b1.reshape(1, 10), W2, b2.reshape(1, 10), W3,
      b3.reshape(1, D))


def _gather_body(x_hbm, table_hbm, out_hbm, x_v, table_v, out_v):
    wid = lax.axis_index("s") * _NC + lax.axis_index("c")
    base = wid * _BPW
    pltpu.sync_copy(x_hbm.at[pl.ds(base, _BPW)], x_v)
    pltpu.sync_copy(table_hbm, table_v)
    row_iota = lax.iota(jnp.int32, _L)

    def body(g, carry):
        xv = x_v[pl.ds(g * _L, _L)]
        src_base = xv * D
        dst_base = (g * _L + row_iota) * D
        for j in range(D):
            vals = plsc.load_gather(table_v, [src_base + j])
            plsc.store_scatter(out_v, [dst_base + j], vals)
        return carry

    lax.fori_loop(0, _GRP, body, 0)
    pltpu.sync_copy(out_v, out_hbm.at[pl.ds(base * D, _BPW * D)])


@functools.partial(
    pl.kernel,
    mesh=plsc.VectorSubcoreMesh(core_axis_name="c", subcore_axis_name="s"),
    compiler_params=pltpu.CompilerParams(needs_layout_passes=False),
    out_type=jax.ShapeDtypeStruct((B * D,), jnp.float32),
    scratch_types=[
        pltpu.VMEM((_BPW,), jnp.int32),
        pltpu.VMEM((V * D,), jnp.float32),
        pltpu.VMEM((_BPW * D,), jnp.float32),
    ],
)
def _sc_gather(x_hbm, table_hbm, out_hbm, x_v, table_v, out_v):
    _gather_body(x_hbm, table_hbm, out_hbm, x_v, table_v, out_v)


def kernel(x, emb, W1, b1, W2, b2, W3, b3):
    table = _compute_table(emb, W1, b1, W2, b2, W3, b3)
    out_flat = _sc_gather(x.astype(jnp.int32), table.reshape(V * D))
    return out_flat.reshape(B, D)
